# Initial kernel scaffold; baseline (speedup 1.0000x reference)
import functools

import jax
import jax.numpy as jnp
from jax import lax
from jax.experimental import pallas as pl
from jax.experimental.pallas import tpu as pltpu
from jax.experimental.pallas import tpu_sc as plsc

VOCAB = 1000000
D = 64
B_TOT = 16384 * 50
NC, NS = 2, 16
NW = NC * NS
CHUNK = 128
ROWS_PER_W = B_TOT // NW
CHUNKS_PER_W = ROWS_PER_W // CHUNK


def _make_gather():
    mesh = plsc.VectorSubcoreMesh(core_axis_name="c", subcore_axis_name="s")

    @functools.partial(
        pl.kernel,
        mesh=mesh,
        out_type=jax.ShapeDtypeStruct((B_TOT, D), jnp.float32),
        scratch_types=[
            pltpu.VMEM((CHUNKS_PER_W, CHUNK), jnp.int32),
            pltpu.VMEM((CHUNK, D), jnp.float32),
            pltpu.SemaphoreType.DMA,
        ],
    )
    def gather_kernel(idx_hbm, table_hbm, out_hbm, idx_v, rows_v, sem):
        wid = lax.axis_index("s") * NC + lax.axis_index("c")
        pltpu.sync_copy(idx_hbm.at[pl.ds(wid * CHUNKS_PER_W, CHUNKS_PER_W)],
                        idx_v)
        base_row = wid * ROWS_PER_W

        def body(s, carry):
            pltpu.async_copy(table_hbm.at[idx_v.at[s]], rows_v, sem).wait()
            pltpu.sync_copy(rows_v,
                            out_hbm.at[pl.ds(base_row + s * CHUNK, CHUNK)])
            return carry

        lax.fori_loop(0, CHUNKS_PER_W, body, 0)

    return gather_kernel


_gather = _make_gather()


@jax.jit
def kernel(x, table):
    idx = x.reshape(B_TOT // CHUNK, CHUNK).astype(jnp.int32)
    out = _gather(idx, table)
    return out.reshape(x.shape[0], x.shape[1], D)


# speedup vs baseline: 1.6846x; 1.6846x over previous
import functools

import jax
import jax.numpy as jnp
from jax import lax
from jax.experimental import pallas as pl
from jax.experimental.pallas import tpu as pltpu
from jax.experimental.pallas import tpu_sc as plsc

VOCAB = 1000000
D = 64
B_TOT = 16384 * 50
NC, NS = 2, 16
NW = NC * NS
CHUNK = 128
ROWS_PER_W = B_TOT // NW
CHUNKS_PER_W = ROWS_PER_W // CHUNK


def _make_gather():
    mesh = plsc.VectorSubcoreMesh(core_axis_name="c", subcore_axis_name="s")

    @functools.partial(
        pl.kernel,
        mesh=mesh,
        out_type=jax.ShapeDtypeStruct((B_TOT, D), jnp.float32),
        compiler_params=pltpu.CompilerParams(use_tc_tiling_on_sc=False),
        scratch_types=[
            pltpu.VMEM((CHUNKS_PER_W, CHUNK), jnp.int32),
            pltpu.VMEM((CHUNK, D), jnp.float32),
            pltpu.SemaphoreType.DMA,
        ],
    )
    def gather_kernel(idx_hbm, table_hbm, out_hbm, idx_v, rows_v, sem):
        wid = lax.axis_index("s") * NC + lax.axis_index("c")
        pltpu.sync_copy(idx_hbm.at[pl.ds(wid * CHUNKS_PER_W, CHUNKS_PER_W)],
                        idx_v)
        base_row = wid * ROWS_PER_W

        def body(s, carry):
            pltpu.async_copy(table_hbm.at[idx_v.at[s]], rows_v, sem).wait()
            pltpu.sync_copy(rows_v,
                            out_hbm.at[pl.ds(base_row + s * CHUNK, CHUNK)])
            return carry

        lax.fori_loop(0, CHUNKS_PER_W, body, 0)

    return gather_kernel


_gather = _make_gather()


@jax.jit
def kernel(x, table):
    idx = x.reshape(B_TOT // CHUNK, CHUNK).astype(jnp.int32)
    out = _gather(idx, table)
    return out.reshape(x.shape[0], x.shape[1], D)


# speedup vs baseline: 1.8742x; 1.1125x over previous
import functools

import jax
import jax.numpy as jnp
from jax import lax
from jax.experimental import pallas as pl
from jax.experimental.pallas import tpu as pltpu
from jax.experimental.pallas import tpu_sc as plsc

VOCAB = 1000000
D = 64
B_TOT = 16384 * 50
NC, NS = 2, 16
NW = NC * NS
CHUNK = 128
ROWS_PER_W = B_TOT // NW
CHUNKS_PER_W = ROWS_PER_W // CHUNK
K = 5
SUPER = K * CHUNK
NSUP = CHUNKS_PER_W // K


def _make_gather():
    mesh = plsc.VectorSubcoreMesh(core_axis_name="c", subcore_axis_name="s")

    @functools.partial(
        pl.kernel,
        mesh=mesh,
        out_type=jax.ShapeDtypeStruct((B_TOT, D), jnp.float32),
        compiler_params=pltpu.CompilerParams(use_tc_tiling_on_sc=False),
        scratch_types=[
            pltpu.VMEM((CHUNKS_PER_W, CHUNK), jnp.int32),
            pltpu.VMEM((SUPER, D), jnp.float32),
            pltpu.VMEM((SUPER, D), jnp.float32),
            pltpu.SemaphoreType.DMA,
            pltpu.SemaphoreType.DMA,
        ],
    )
    def gather_kernel(idx_hbm, table_hbm, out_hbm, idx_v, buf0, buf1,
                      gsem0, gsem1):
        wid = lax.axis_index("s") * NC + lax.axis_index("c")
        pltpu.sync_copy(idx_hbm.at[pl.ds(wid * CHUNKS_PER_W, CHUNKS_PER_W)],
                        idx_v)
        base_row = wid * ROWS_PER_W

        def fire(sc, buf, sem):
            for j in range(K):
                pltpu.async_copy(
                    table_hbm.at[idx_v.at[sc * K + j]],
                    buf.at[pl.ds(j * CHUNK, CHUNK)], sem)

        def drain(sc, buf, sem):
            for j in range(K):
                pltpu.make_async_copy(
                    table_hbm.at[idx_v.at[sc * K + j]],
                    buf.at[pl.ds(j * CHUNK, CHUNK)], sem).wait()

        def store(sc, buf):
            pltpu.sync_copy(buf, out_hbm.at[pl.ds(base_row + sc * SUPER,
                                                  SUPER)])

        fire(0, buf0, gsem0)

        def body(t, carry):
            sc0 = 2 * t
            fire(sc0 + 1, buf1, gsem1)
            drain(sc0, buf0, gsem0)
            store(sc0, buf0)
            fire(sc0 + 2, buf0, gsem0)
            drain(sc0 + 1, buf1, gsem1)
            store(sc0 + 1, buf1)
            return carry

        lax.fori_loop(0, NSUP // 2 - 1, body, 0)

        fire(NSUP - 1, buf1, gsem1)
        drain(NSUP - 2, buf0, gsem0)
        store(NSUP - 2, buf0)
        drain(NSUP - 1, buf1, gsem1)
        store(NSUP - 1, buf1)

    return gather_kernel


_gather = _make_gather()


@jax.jit
def kernel(x, table):
    idx = x.reshape(B_TOT // CHUNK, CHUNK).astype(jnp.int32)
    out = _gather(idx, table)
    return out.reshape(x.shape[0], x.shape[1], D)
